# baseline (speedup 1.0000x reference)
import functools

import numpy as np
import jax
import jax.numpy as jnp
from jax import lax
from jax.experimental import pallas as pl
from jax.experimental.pallas import tpu as pltpu
from jax.experimental.pallas import tpu_sc as plsc

F = 26
K = 16
N = 100000
B = 4096
NC = 2
NS = 16
NW = NC * NS
NP = F * K
PPW = NP // NW
NPAIR = (F * (F - 1) // 2) * K
NROW = 2 * NPAIR
TRASH = NROW
LROW = NROW + 1
WROWS = NROW + 2
TB = B // K
BPW = B // NW
CH2 = 208
NCH2 = NROW // CH2


def _dest_table():
    slot = {}
    q = 0
    for i in range(F):
        for j in range(i + 1, F):
            slot[(i, j)] = q
            q += 1
    dest = np.full((NP * F, 16), TRASH, np.int32)
    for p in range(NP):
        j, k = p // K, p % K
        for g in range(F):
            if g < j:
                d = 2 * (slot[(g, j)] * K + k)
            elif g > j:
                d = 2 * (slot[(j, g)] * K + k) + 1
            else:
                d = TRASH
            dest[p * F + g, 0] = d
    return dest


def _phase1():
    mesh = plsc.VectorSubcoreMesh(core_axis_name="c", subcore_axis_name="s",
                                  num_cores=NC, num_subcores=NS)

    @functools.partial(
        pl.kernel,
        mesh=mesh,
        compiler_params=pltpu.CompilerParams(needs_layout_passes=False,
                                             use_tc_tiling_on_sc=False),
        out_type=jax.ShapeDtypeStruct((WROWS, TB, K), jnp.float32),
        scratch_types=[
            pltpu.VMEM((N,), jnp.float32),
            pltpu.VMEM((2, TB, K), jnp.int32),
            pltpu.VMEM((2, TB, K), jnp.float32),
            pltpu.VMEM((PPW * F, K), jnp.int32),
            pltpu.VMEM((K,), jnp.float32),
            pltpu.VMEM_SHARED((F, TB, K), jnp.int32),
            pltpu.SemaphoreType.DMA,
            pltpu.SemaphoreType.DMA,
        ],
    )
    def body(xt_hbm, vt_hbm, lt_hbm, dest_hbm, b_hbm, w_hbm,
             plane, idx, rows, destb, bv, xs, isem, osem):
        sub = lax.axis_index("s")
        wid = sub * NC + lax.axis_index("c")

        @pl.when(sub == 0)
        def _():
            pltpu.sync_copy(xt_hbm, xs)
        plsc.subcore_barrier()

        pltpu.sync_copy(dest_hbm.at[pl.ds(wid * (PPW * F), PPW * F)], destb)
        pltpu.sync_copy(b_hbm, bv.at[pl.ds(0, 1)])

        pltpu.async_copy(xs.at[0], idx.at[0], isem)

        def iteration(ii, carry):
            g = lax.rem(ii, F)
            par = lax.rem(ii, 2)

            @pl.when(g == 0)
            def _():
                pltpu.sync_copy(vt_hbm.at[wid * PPW + ii // F], plane)

            @pl.when(ii >= 2)
            def _():
                prow = destb[ii - 2, pl.ds(0, K)][0]
                pltpu.make_async_copy(rows.at[par],
                                      w_hbm.at[prow], osem).wait()
            pltpu.make_async_copy(xs.at[g], idx.at[par], isem).wait()

            @pl.when(ii + 1 < PPW * F)
            def _():
                pltpu.async_copy(xs.at[lax.rem(ii + 1, F)],
                                 idx.at[lax.rem(ii + 1, 2)], isem)

            def step(t, c2):
                iv = idx[par, t, pl.ds(0, K)]
                rows[par, t, pl.ds(0, K)] = plsc.load_gather(plane, [iv])
                return c2

            lax.fori_loop(0, TB, step, 0)
            drow = destb[ii, pl.ds(0, K)][0]
            pltpu.async_copy(rows.at[par], w_hbm.at[drow], osem)
            return carry

        lax.fori_loop(0, PPW * F, iteration, 0)
        for pii in (PPW * F - 2, PPW * F - 1):
            prow = destb[pii, pl.ds(0, K)][0]
            pltpu.make_async_copy(rows.at[pii % 2],
                                  w_hbm.at[prow], osem).wait()

        @pl.when(wid == NW - 1)
        def _():
            pltpu.sync_copy(lt_hbm, plane)
            b0 = bv[pl.ds(0, K)][0]

            def init(t, carry):
                rows[0, t, pl.ds(0, K)] = jnp.full((K,), b0, jnp.float32)
                return carry
            lax.fori_loop(0, TB, init, 0)

            def lstep(t, carry):
                iv = idx[0, t, pl.ds(0, K)]
                rows[0, t, pl.ds(0, K)] = (rows[0, t, pl.ds(0, K)]
                                           + plsc.load_gather(plane, [iv]))
                return carry

            for g in range(F):
                pltpu.sync_copy(xs.at[g], idx.at[0])
                lax.fori_loop(0, TB, lstep, 0)
            pltpu.sync_copy(rows.at[0], w_hbm.at[LROW])

    return body


def _phase2():
    mesh = plsc.VectorSubcoreMesh(core_axis_name="c", subcore_axis_name="s",
                                  num_cores=NC, num_subcores=NS)
    NVB = BPW // K

    @functools.partial(
        pl.kernel,
        mesh=mesh,
        compiler_params=pltpu.CompilerParams(needs_layout_passes=False,
                                             use_tc_tiling_on_sc=False),
        out_type=jax.ShapeDtypeStruct((B,), jnp.float32),
        scratch_types=[
            pltpu.VMEM((2, CH2), jnp.int32),
            pltpu.VMEM((2, CH2, BPW), jnp.float32),
            pltpu.VMEM((BPW,), jnp.float32),
            pltpu.SemaphoreType.DMA,
        ],
    )
    def body(w_hbm, out_hbm, idx, buf, obuf, sem):
        wid = lax.axis_index("s") * NC + lax.axis_index("c")
        lane = lax.iota(jnp.int32, K)

        def build(c, par):
            base = c * CH2
            for m in range(CH2 // K):
                idx[par, pl.ds(m * K, K)] = (base + m * K + lane) * NW + wid

        def fire(par):
            pltpu.async_copy(w_hbm.at[idx.at[par]], buf.at[par], sem)

        def drain(par):
            pltpu.make_async_copy(w_hbm.at[idx.at[par]], buf.at[par],
                                  sem).wait()

        build(0, 0)
        fire(0)

        def chunk(c, accs):
            par = c % 2

            @pl.when(c + 1 < NCH2)
            def _():
                build(c + 1, (c + 1) % 2)
            drain(par)

            @pl.when(c + 1 < NCH2)
            def _():
                fire((c + 1) % 2)

            def pair(q, accs):
                return tuple(
                    accs[m] + buf[par, 2 * q, pl.ds(m * K, K)]
                    * buf[par, 2 * q + 1, pl.ds(m * K, K)]
                    for m in range(NVB))

            return lax.fori_loop(0, CH2 // 2, pair, accs)

        accs = tuple(jnp.zeros((K,), jnp.float32) for _ in range(NVB))
        accs = lax.fori_loop(0, NCH2, chunk, accs)

        pltpu.sync_copy(w_hbm.at[LROW * NW + wid], obuf)
        for m in range(NVB):
            obuf[pl.ds(m * K, K)] = accs[m] + obuf[pl.ds(m * K, K)]
        pltpu.sync_copy(obuf, out_hbm.at[pl.ds(wid * BPW, BPW)])

    return body


@jax.jit
def kernel(x, linear_w, v_w, b):
    vt = v_w.transpose(1, 2, 0).reshape(NP, N)
    xt = x.T.reshape(F, TB, K)
    lt = linear_w.reshape(N)
    dest = jnp.asarray(_dest_table())
    w = _phase1()(xt, vt, lt, dest, b)
    out = _phase2()(w.reshape(WROWS * NW, BPW))
    return out.reshape(B, 1)
